# Initial kernel scaffold; baseline (speedup 1.0000x reference)
import jax
import jax.numpy as jnp
from jax.experimental import pallas as pl


def kernel(x, edge_index, W_enc0, b_enc0, W_enc1, b_enc1, W_str0, b_str0, W_str1, b_str1, W_att0, b_att0, W_att1, b_att1):
    raise NotImplementedError("write your pallas kernel here")


# speedup vs baseline: 7.6628x; 7.6628x over previous
import functools

import jax
import jax.numpy as jnp
from jax import lax
from jax.experimental import pallas as pl
from jax.experimental.pallas import tpu as pltpu
from jax.experimental.pallas import tpu_sc as plsc

N = 10000
D_IN = 128
D_HID = 64
E = 320000

NCORES = 2
NSUB = 16
NTILES = NCORES * NSUB
CHUNK = 512
EDGES_PER_TILE = 10240
NCHUNK = EDGES_PER_TILE // CHUNK
EPAD = NTILES * EDGES_PER_TILE
NACC = N + 8
ROWS_A = 624
ROWS_B = N - 15 * ROWS_A

F32 = jnp.float32


def _make_scatter_add(width):
  mesh = plsc.VectorSubcoreMesh(core_axis_name="c", subcore_axis_name="s")

  @functools.partial(
      pl.kernel,
      out_type=jax.ShapeDtypeStruct((NCORES, N, width), F32),
      mesh=mesh,
      scratch_types=[
          pltpu.VMEM_SHARED((NACC, width), F32),
          pltpu.VMEM((CHUNK,), jnp.int32),
          pltpu.VMEM((CHUNK,), jnp.int32),
          pltpu.VMEM((CHUNK, width), F32),
          pltpu.SemaphoreType.DMA,
      ],
      compiler_params=pltpu.CompilerParams(use_tc_tiling_on_sc=False),
  )
  def scatter_add(u_hbm, src_hbm, dst_hbm, out_hbm,
                  accum, src_v, dst_v, rows_v, sem):
    c = lax.axis_index("c")
    s = lax.axis_index("s")
    wid = c * NSUB + s
    row0 = s * ROWS_A

    @pl.when(s < NSUB - 1)
    def _():
      pltpu.sync_copy(u_hbm.at[pl.ds(row0, ROWS_A)],
                      accum.at[pl.ds(row0, ROWS_A)])

    @pl.when(s == NSUB - 1)
    def _():
      pltpu.sync_copy(u_hbm.at[pl.ds(15 * ROWS_A, ROWS_B)],
                      accum.at[pl.ds(15 * ROWS_A, ROWS_B)])

    plsc.subcore_barrier()

    base = wid * EDGES_PER_TILE

    def body(j, carry):
      off = base + j * CHUNK
      pltpu.sync_copy(src_hbm.at[pl.ds(off, CHUNK)], src_v)
      pltpu.sync_copy(dst_hbm.at[pl.ds(off, CHUNK)], dst_v)
      pltpu.async_copy(u_hbm.at[src_v], rows_v, sem).wait()
      pltpu.sync_copy(rows_v, accum.at[dst_v], add=True)
      return carry

    lax.fori_loop(0, NCHUNK, body, 0)
    plsc.subcore_barrier()

    @pl.when(s < NSUB - 1)
    def _():
      pltpu.sync_copy(accum.at[pl.ds(row0, ROWS_A)],
                      out_hbm.at[c, pl.ds(row0, ROWS_A)])

    @pl.when(s == NSUB - 1)
    def _():
      pltpu.sync_copy(accum.at[pl.ds(15 * ROWS_A, ROWS_B)],
                      out_hbm.at[c, pl.ds(15 * ROWS_A, ROWS_B)])

  return scatter_add


_scatter64 = _make_scatter_add(D_HID)
_scatter16 = _make_scatter_add(16)


BR = 1000
GRID = N // BR


def _row_spec(w):
  return pl.BlockSpec((BR, w), lambda i: (i, 0))


def _full_spec(r, c):
  return pl.BlockSpec((r, c), lambda i: (0, 0))


def _deg_body(d0, d1, o):
  o[...] = lax.rsqrt(d0[...][:, :1] + d1[...][:, :1] - 1.0)


def _tc_deg(d0, d1):
  return pl.pallas_call(
      _deg_body,
      grid=(GRID,),
      in_specs=[_row_spec(16), _row_spec(16)],
      out_specs=pl.BlockSpec((BR, 1), lambda i: (i, 0)),
      out_shape=jax.ShapeDtypeStruct((N, 1), F32),
  )(d0, d1)


def _pre_body(x, w, dinv, o):
  o[...] = dinv[...] * jnp.dot(x[...], w[...], preferred_element_type=F32)


def _tc_pre(x, w, dinv):
  return pl.pallas_call(
      _pre_body,
      grid=(GRID,),
      in_specs=[_row_spec(D_IN), _full_spec(D_IN, D_HID), _row_spec(1)],
      out_specs=_row_spec(D_HID),
      out_shape=jax.ShapeDtypeStruct((N, D_HID), F32),
  )(x, w, dinv)


def _finish(q0, q1, u, b, dinv):
  return jnp.maximum(dinv[...] * (q0[...] + q1[...] - u[...]) + b[...], 0.0)


def _mid_body(q0, q1, u, b, w, dinv, o):
  z = _finish(q0, q1, u, b, dinv)
  o[...] = dinv[...] * jnp.dot(z, w[...], preferred_element_type=F32)


def _tc_mid(q, u, b, w, dinv):
  return pl.pallas_call(
      _mid_body,
      grid=(GRID,),
      in_specs=[_row_spec(D_HID), _row_spec(D_HID), _row_spec(D_HID),
                _full_spec(1, D_HID), _full_spec(D_HID, D_HID), _row_spec(1)],
      out_specs=_row_spec(D_HID),
      out_shape=jax.ShapeDtypeStruct((N, D_HID), F32),
  )(q[0], q[1], u, b, w, dinv)


def _two_body(q0, q1, u, b, wa, wb, dinv, oa, ob):
  z = _finish(q0, q1, u, b, dinv)
  oa[...] = dinv[...] * jnp.dot(z, wa[...], preferred_element_type=F32)
  ob[...] = dinv[...] * jnp.dot(z, wb[...], preferred_element_type=F32)


def _tc_two(q, u, b, wa, wb, dinv):
  return pl.pallas_call(
      _two_body,
      grid=(GRID,),
      in_specs=[_row_spec(D_HID), _row_spec(D_HID), _row_spec(D_HID),
                _full_spec(1, D_HID), _full_spec(D_HID, D_HID),
                _full_spec(D_HID, D_HID), _row_spec(1)],
      out_specs=(_row_spec(D_HID), _row_spec(D_HID)),
      out_shape=(jax.ShapeDtypeStruct((N, D_HID), F32),
                 jax.ShapeDtypeStruct((N, D_HID), F32)),
  )(q[0], q[1], u, b, wa, wb, dinv)


def _fin_body(q0, q1, u, b, dinv, o):
  o[...] = _finish(q0, q1, u, b, dinv)


def _tc_fin(q, u, b, dinv):
  return pl.pallas_call(
      _fin_body,
      grid=(GRID,),
      in_specs=[_row_spec(D_HID), _row_spec(D_HID), _row_spec(D_HID),
                _full_spec(1, D_HID), _row_spec(1)],
      out_specs=_row_spec(D_HID),
      out_shape=jax.ShapeDtypeStruct((N, D_HID), F32),
  )(q[0], q[1], u, b, dinv)


def _nw_body(q0, q1, u, b, dinv, o):
  o[...] = dinv[...] * _finish(q0, q1, u, b, dinv)


def _tc_nw(q, u, b, dinv):
  return pl.pallas_call(
      _nw_body,
      grid=(GRID,),
      in_specs=[_row_spec(D_HID), _row_spec(D_HID), _row_spec(D_HID),
                _full_spec(1, D_HID), _row_spec(1)],
      out_specs=_row_spec(D_HID),
      out_shape=jax.ShapeDtypeStruct((N, D_HID), F32),
  )(q[0], q[1], u, b, dinv)


def _att_body(q0, q1, u, b, w, dinv, o):
  ph = dinv[...] * (q0[...] + q1[...] - u[...])
  o[...] = jnp.maximum(
      jnp.dot(ph, w[...], preferred_element_type=F32) + b[...], 0.0)


def _tc_att(q, u, b, w, dinv):
  return pl.pallas_call(
      _att_body,
      grid=(GRID,),
      in_specs=[_row_spec(D_HID), _row_spec(D_HID), _row_spec(D_HID),
                _full_spec(1, D_IN), _full_spec(D_HID, D_IN), _row_spec(1)],
      out_specs=_row_spec(D_IN),
      out_shape=jax.ShapeDtypeStruct((N, D_IN), F32),
  )(q[0], q[1], u, b, w, dinv)


def _sst_body(a, b, o):
  o[...] = lax.dot_general(a[...], b[...], (((1,), (1,)), ((), ())),
                           preferred_element_type=F32)


CB = 1024


def _tc_sst(s):
  return pl.pallas_call(
      _sst_body,
      grid=(GRID, pl.cdiv(N, CB)),
      in_specs=[pl.BlockSpec((BR, D_HID), lambda i, j: (i, 0)),
                pl.BlockSpec((CB, D_HID), lambda i, j: (j, 0))],
      out_specs=pl.BlockSpec((BR, CB), lambda i, j: (i, j)),
      out_shape=jax.ShapeDtypeStruct((N, N), F32),
  )(s, s)


def kernel(x, edge_index, W_enc0, b_enc0, W_enc1, b_enc1, W_str0, b_str0,
           W_str1, b_str1, W_att0, b_att0, W_att1, b_att1):
  pad = EPAD - E
  src = jnp.concatenate([edge_index[0], jnp.zeros((pad,), jnp.int32)])
  dst = jnp.concatenate([edge_index[1], jnp.full((pad,), N, jnp.int32)])

  b_enc0 = b_enc0.reshape(1, -1)
  b_enc1 = b_enc1.reshape(1, -1)
  b_str0 = b_str0.reshape(1, -1)
  b_str1 = b_str1.reshape(1, -1)
  b_att0 = b_att0.reshape(1, -1)
  b_att1 = b_att1.reshape(1, -1)

  ones16 = jnp.ones((N, 16), F32)
  dpart = _scatter16(ones16, src, dst)
  dinv = _tc_deg(dpart[0], dpart[1])

  u1 = _tc_pre(x, W_enc0, dinv)
  q1 = _scatter64(u1, src, dst)
  u2 = _tc_mid(q1, u1, b_enc0, W_enc1, dinv)
  q2 = _scatter64(u2, src, dst)
  u3, u5 = _tc_two(q2, u2, b_enc1, W_str0, W_att0, dinv)
  q3 = _scatter64(u3, src, dst)
  u4 = _tc_mid(q3, u3, b_str0, W_str1, dinv)
  q4 = _scatter64(u4, src, dst)
  s = _tc_fin(q4, u4, b_str1, dinv)
  stru_recon = _tc_sst(s)
  q5 = _scatter64(u5, src, dst)
  u6 = _tc_nw(q5, u5, b_att0, dinv)
  q6 = _scatter64(u6, src, dst)
  attr_recon = _tc_att(q6, u6, b_att1, W_att1, dinv)
  return (stru_recon, attr_recon)
